# Initial kernel scaffold; baseline (speedup 1.0000x reference)
import jax
import jax.numpy as jnp
from jax import lax
from jax.experimental import pallas as pl
from jax.experimental.pallas import tpu as pltpu
from jax.experimental.pallas import tpu_sc as plsc

B = 1024
F = 26
T = 26
D = 32
V = 1000 * F
NC = 2
NS = 16
NW = NC * NS

PAIRS = B * F
PAIRS_PER_W = PAIRS // NW
ROWS_PER_W = PAIRS_PER_W * T
CHUNK_PAIRS = 32
CHUNK_ROWS = CHUNK_PAIRS * T
NCHUNK = PAIRS_PER_W // CHUNK_PAIRS
N_ROWS = PAIRS * T


def _sc_body(tab_hbm, x_hbm, out_hbm, xv, bv, idxv, rows0, rows1,
             gsem, wsem0, wsem1):
    w = lax.axis_index("s") * NC + lax.axis_index("c")
    lane = lax.iota(jnp.int32, 16)

    pltpu.sync_copy(x_hbm.at[pl.ds(w * PAIRS_PER_W, PAIRS_PER_W)], xv)

    def bv_body(j, carry):
        p16 = j * 16 + lane
        f16 = lax.rem(p16, F)
        bv[pl.ds(j * 16, 16)] = xv[pl.ds(j * 16, 16)] + f16 * 1000
        return carry

    lax.fori_loop(0, PAIRS_PER_W // 16, bv_body, 0)

    def idx_body(i, carry):
        r16 = i * 16 + lane
        p16 = lax.div(r16, T)
        t16 = r16 - p16 * T
        b16 = plsc.load_gather(bv, [p16])
        idxv[pl.ds(i * 16, 16)] = b16 + t16 * V
        return carry

    lax.fori_loop(0, ROWS_PER_W // 16, idx_body, 0)

    rows = (rows0, rows1)
    wsems = (wsem0, wsem1)
    out0 = w * ROWS_PER_W

    for c in range(NCHUNK):
        bb = c % 2
        if c >= 2:
            pltpu.make_async_copy(
                rows[bb], out_hbm.at[pl.ds(0, CHUNK_ROWS)], wsems[bb]).wait()
        pltpu.async_copy(
            tab_hbm.at[idxv.at[pl.ds(c * CHUNK_ROWS, CHUNK_ROWS)]],
            rows[bb], gsem).wait()
        pltpu.async_copy(
            rows[bb], out_hbm.at[pl.ds(out0 + c * CHUNK_ROWS, CHUNK_ROWS)],
            wsems[bb])

    for bb in range(2):
        pltpu.make_async_copy(
            rows[bb], out_hbm.at[pl.ds(0, CHUNK_ROWS)], wsems[bb]).wait()


@jax.jit
def _sc_gather(tab_flat, x_flat):
    mesh = plsc.VectorSubcoreMesh(
        core_axis_name="c", subcore_axis_name="s",
        num_cores=NC, num_subcores=NS)
    run = pl.kernel(
        _sc_body,
        out_type=jax.ShapeDtypeStruct((N_ROWS, D), jnp.float32),
        mesh=mesh,
        scratch_types=[
            pltpu.VMEM((PAIRS_PER_W,), jnp.int32),
            pltpu.VMEM((PAIRS_PER_W,), jnp.int32),
            pltpu.VMEM((ROWS_PER_W,), jnp.int32),
            pltpu.VMEM((CHUNK_ROWS, D), jnp.float32),
            pltpu.VMEM((CHUNK_ROWS, D), jnp.float32),
            pltpu.SemaphoreType.DMA,
            pltpu.SemaphoreType.DMA,
            pltpu.SemaphoreType.DMA,
        ],
    )
    return run(tab_flat, x_flat)


def kernel(x, tables):
    tab_flat = tables.reshape(T * V, D)
    x_flat = x.reshape(PAIRS)
    out = _sc_gather(tab_flat, x_flat)
    return out.reshape(B, F, T, D)


# speedup vs baseline: 1.7458x; 1.7458x over previous
import jax
import jax.numpy as jnp
from jax import lax
from jax.experimental import pallas as pl
from jax.experimental.pallas import tpu as pltpu
from jax.experimental.pallas import tpu_sc as plsc

B = 1024
F = 26
T = 26
D = 32
V = 1000 * F
NC = 2
NS = 16
NW = NC * NS

PAIRS = B * F
PAIRS_PER_W = PAIRS // NW
HALF = PAIRS_PER_W // 2


def _sc_body(tab_hbm, x_hbm, out_hbm, xv, bv, idx0, idx1, rows0, rows1,
             gsem, wsem0, wsem1):
    w = lax.axis_index("s") * NC + lax.axis_index("c")
    lane = lax.iota(jnp.int32, 16)

    pltpu.sync_copy(x_hbm.at[pl.ds(w * PAIRS_PER_W, PAIRS_PER_W)], xv)

    def bv_body(j, carry):
        p16 = j * 16 + lane
        f16 = lax.rem(p16, F)
        bv[pl.ds(j * 16, 16)] = xv[pl.ds(j * 16, 16)] + f16 * 1000
        return carry

    lax.fori_loop(0, PAIRS_PER_W // 16, bv_body, 0)

    idxs = (idx0, idx1)
    rows = (rows0, rows1)
    wsems = (wsem0, wsem1)
    p0 = w * PAIRS_PER_W

    for t in range(T):
        it = t % 2
        idxv = idxs[it]

        def idx_body(j, carry):
            idxv[pl.ds(j * 16, 16)] = bv[pl.ds(j * 16, 16)] + t * V
            return carry

        lax.fori_loop(0, PAIRS_PER_W // 16, idx_body, 0)

        for h in range(2):
            if t >= 1:
                pltpu.make_async_copy(
                    rows[h],
                    out_hbm.at[pl.ds(0, HALF), pl.ds(0, 1)],
                    wsems[h]).wait()
            pltpu.async_copy(
                tab_hbm.at[idxv.at[pl.ds(h * HALF, HALF)]],
                rows[h], gsem).wait()
            pltpu.async_copy(
                rows[h],
                out_hbm.at[pl.ds(p0 + h * HALF, HALF), pl.ds(t, 1)],
                wsems[h])

    for h in range(2):
        pltpu.make_async_copy(
            rows[h],
            out_hbm.at[pl.ds(0, HALF), pl.ds(0, 1)],
            wsems[h]).wait()


@jax.jit
def _sc_gather(tab3, x_flat):
    mesh = plsc.VectorSubcoreMesh(
        core_axis_name="c", subcore_axis_name="s",
        num_cores=NC, num_subcores=NS)
    run = pl.kernel(
        _sc_body,
        out_type=jax.ShapeDtypeStruct((PAIRS, T, D), jnp.float32),
        mesh=mesh,
        compiler_params=pltpu.CompilerParams(use_tc_tiling_on_sc=False),
        scratch_types=[
            pltpu.VMEM((PAIRS_PER_W,), jnp.int32),
            pltpu.VMEM((PAIRS_PER_W,), jnp.int32),
            pltpu.VMEM((PAIRS_PER_W,), jnp.int32),
            pltpu.VMEM((PAIRS_PER_W,), jnp.int32),
            pltpu.VMEM((HALF, 1, D), jnp.float32),
            pltpu.VMEM((HALF, 1, D), jnp.float32),
            pltpu.SemaphoreType.DMA,
            pltpu.SemaphoreType.DMA,
            pltpu.SemaphoreType.DMA,
        ],
    )
    return run(tab3, x_flat)


def kernel(x, tables):
    tab3 = tables.reshape(T * V, 1, D)
    x_flat = x.reshape(PAIRS)
    out = _sc_gather(tab3, x_flat)
    return out.reshape(B, F, T, D)
